# Initial kernel scaffold; baseline (speedup 1.0000x reference)
import jax
import jax.numpy as jnp
from jax.experimental import pallas as pl


def kernel(user_idx, user_features, user_color_idx, user_size_idx, item_idx, item_features, params):
    raise NotImplementedError("write your pallas kernel here")


# speedup vs baseline: 3.0666x; 3.0666x over previous
import functools

import jax
import jax.numpy as jnp
from jax import lax
from jax.experimental import pallas as pl
from jax.experimental.pallas import tpu as pltpu

_B = 32768
_E = 64
_BB = 512
_LN_EPS = 1e-5
_BN_EPS = 1e-5

_F32 = jnp.float32
_BF16 = jnp.bfloat16


def _ln(x, g, b):
    m = x.mean(-1, keepdims=True)
    v = jnp.var(x, axis=-1, keepdims=True)
    return (x - m) * lax.rsqrt(v + _LN_EPS) * g + b


def _dot(x, w):
    return jnp.dot(x.astype(_BF16), w, preferred_element_type=_F32)


def _body(idx_ref, emb_ref, feat_ref,
          ifw_ref, cae_ref, caf_ref, cao_ref,
          vw_ref, ow_ref, fuw_ref,
          fi1w1_ref, fi1w2_ref, fi2w1_ref, fi2w2_ref,
          pw1_ref, pw2_ref, pw3_ref,
          v64_ref, v256_ref, v128_ref, c4_ref,
          out_ref, tile_ref):
    base = pl.program_id(0) * _BB
    iota8 = lax.broadcasted_iota(jnp.int32, (8, _E), 0)

    def gather8(o, _):
        rows = []
        for k in range(8):
            idx = idx_ref[base + o * 8 + k]
            cbase = pl.multiple_of((idx >> 3) << 3, 8)
            chunk = emb_ref[pl.ds(cbase, 8), :]
            sel = (iota8 == (idx & 7)).astype(_F32)
            rows.append(jnp.sum(chunk * sel, axis=0, keepdims=True))
        tile_ref[pl.ds(pl.multiple_of(o * 8, 8), 8), :] = jnp.concatenate(rows, axis=0)
        return 0

    lax.fori_loop(0, _BB // 8, gather8, 0)

    v = lambda r: v64_ref[r:r + 1, :]
    if_b, ca_ib, ca_ob, fusion_b = v(0), v(1), v(2), v(3)
    msa_g, msa_be = v(4), v(5)
    fi1_g, fi1_be, fi1_b2 = v(6), v(7), v(8)
    fi2_g, fi2_be, fi2_b2 = v(9), v(10), v(11)
    on_g, on_be = v(12), v(13)
    s3, c3, w4 = v(14), v(15), v(16)
    fi1_b1, s1, c1 = v256_ref[0:1, :], v256_ref[1:2, :], v256_ref[2:3, :]
    fi2_b1, s2, c2 = v128_ref[0:1, :], v128_ref[1:2, :], v128_ref[2:3, :]

    emb = tile_ref[...]
    feat = feat_ref[...]
    iflin = jnp.dot(feat, ifw_ref[...], preferred_element_type=_F32) + if_b
    kv = (jnp.dot(emb.astype(_BF16), cae_ref[...], preferred_element_type=_F32)
          + _dot(iflin, caf_ref[...]) + ca_ib)
    ca = _dot(kv, cao_ref[...]) + ca_ob
    fused = jnp.broadcast_to(fusion_b, ca.shape)
    for i in range(3):
        sl = slice(i * _E, (i + 1) * _E)
        vi = _dot(ca, vw_ref[sl, :]) + v(17 + i)
        oi = _dot(vi, ow_ref[sl, :]) + v(20 + i)
        fused = fused + _dot(oi, fuw_ref[sl, :])
    x = _ln(fused + ca, msa_g, msa_be)
    h = _ln(x, fi1_g, fi1_be)
    h = jax.nn.relu(_dot(h, fi1w1_ref[...]) + fi1_b1)
    h = jax.nn.relu(_dot(h, fi1w2_ref[...]) + fi1_b2)
    x = x + h
    h = _ln(x, fi2_g, fi2_be)
    h = jax.nn.relu(_dot(h, fi2w1_ref[...]) + fi2_b1)
    h = jax.nn.relu(_dot(h, fi2w2_ref[...]) + fi2_b2)
    x = x + h
    x = _ln(x, on_g, on_be)
    y = jax.nn.relu(_dot(x, pw1_ref[...]) * s1 + c1)
    y = jax.nn.relu(_dot(y, pw2_ref[...]) * s2 + c2)
    y = jax.nn.relu(_dot(y, pw3_ref[...]) * s3 + c3)
    yb = y.astype(_BF16).astype(_F32)
    w4b = w4.astype(_BF16).astype(_F32)
    out_ref[...] = jnp.sum(yb * w4b, axis=-1, keepdims=True) + c4_ref[0, 0]


@jax.jit
def kernel(user_idx, user_features, user_color_idx, user_size_idx,
           item_idx, item_features, params):
    del user_idx, user_features, user_color_idx, user_size_idx
    p = params

    def bnf(bg, bb, bm, bv):
        s = bg * lax.rsqrt(bv + _BN_EPS)
        return s, (0.0 - bm) * s + bb

    s1, c1 = bnf(p['bn1_g'], p['bn1_b'], p['bn1_m'], p['bn1_v'])
    s2, c2 = bnf(p['bn2_g'], p['bn2_b'], p['bn2_m'], p['bn2_v'])
    s3, c3 = bnf(p['bn3_g'], p['bn3_b'], p['bn3_m'], p['bn3_v'])
    c1 = p['p_b1'] * s1 + c1
    c2 = p['p_b2'] * s2 + c2
    c3 = p['p_b3'] * s3 + c3

    z64 = jnp.zeros((_E,), _F32)
    v64 = jnp.stack([
        p['if_b'], p['ca_ib'], p['ca_ob'], p['fusion_b'],
        p['msa_g'], p['msa_be'],
        p['fi1_g'], p['fi1_be'], p['fi1_b2'],
        p['fi2_g'], p['fi2_be'], p['fi2_b2'],
        p['on_g'], p['on_be'],
        s3, c3, p['p_W4'][0],
        p['msa_inb'][0][2 * _E:], p['msa_inb'][1][2 * _E:], p['msa_inb'][2][2 * _E:],
        p['msa_ob'][0], p['msa_ob'][1], p['msa_ob'][2], z64,
    ], axis=0)
    v256 = jnp.stack([p['fi1_b1'], s1, c1], axis=0)
    v128 = jnp.stack([p['fi2_b1'], s2, c2], axis=0)
    c4 = p['p_b4'].reshape(1, 1)

    bf = lambda a: a.astype(_BF16)
    vw = jnp.concatenate([p['msa_inW'][i][2 * _E:, :].T for i in range(3)], axis=0)
    ow = jnp.concatenate([p['msa_oW'][i].T for i in range(3)], axis=0)
    fuw = jnp.concatenate([p['fusion_W'][:, i * _E:(i + 1) * _E].T for i in range(3)], axis=0)

    idx = item_idx.astype(jnp.int32)
    emb = p['item_emb']
    feat = bf(item_features)

    nb = _B // _BB
    cm = lambda i, s: (0, 0)
    wspec = lambda: pl.BlockSpec(None, cm)
    grid_spec = pltpu.PrefetchScalarGridSpec(
        num_scalar_prefetch=1,
        grid=(nb,),
        in_specs=[
            pl.BlockSpec((emb.shape[0], _E), cm),
            pl.BlockSpec((_BB, 128), lambda i, s: (i, 0)),
            wspec(), wspec(), wspec(), wspec(),
            wspec(), wspec(), wspec(),
            wspec(), wspec(), wspec(), wspec(),
            wspec(), wspec(), wspec(),
            wspec(), wspec(), wspec(), wspec(),
        ],
        out_specs=pl.BlockSpec((_BB, 1), lambda i, s: (i, 0)),
        scratch_shapes=[pltpu.VMEM((_BB, _E), _F32)],
    )
    return pl.pallas_call(
        _body,
        grid_spec=grid_spec,
        out_shape=jax.ShapeDtypeStruct((_B, 1), _F32),
        compiler_params=pltpu.CompilerParams(
            dimension_semantics=("parallel",),
        ),
    )(idx, emb, feat,
      bf(p['if_W'].T), bf(p['ca_iW'][:, :_E].T), bf(p['ca_iW'][:, _E:].T), bf(p['ca_oW'].T),
      bf(vw), bf(ow), bf(fuw),
      bf(p['fi1_W1'].T), bf(p['fi1_W2'].T), bf(p['fi2_W1'].T), bf(p['fi2_W2'].T),
      bf(p['p_W1'].T), bf(p['p_W2'].T), bf(p['p_W3'].T),
      v64, v256, v128, c4)


# speedup vs baseline: 3.3917x; 1.1060x over previous
import functools

import jax
import jax.numpy as jnp
from jax import lax
from jax.experimental import pallas as pl
from jax.experimental.pallas import tpu as pltpu

_B = 32768
_E = 64
_BB = 512
_LN_EPS = 1e-5
_BN_EPS = 1e-5

_F32 = jnp.float32
_BF16 = jnp.bfloat16


def _ln(x, g, b):
    m = x.mean(-1, keepdims=True)
    v = jnp.var(x, axis=-1, keepdims=True)
    return (x - m) * lax.rsqrt(v + _LN_EPS) * g + b


def _dot(x, w):
    return jnp.dot(x.astype(_BF16), w, preferred_element_type=_F32)


def _body(idx_ref, emb_ref, feat_ref,
          ifw_ref, cae_ref, caf_ref, cao_ref,
          vw_ref, ow_ref, fuw_ref,
          fi1w1_ref, fi1w2_ref, fi2w1_ref, fi2w2_ref,
          pw1_ref, pw2_ref, pw3_ref,
          v64_ref, v256_ref, v128_ref, c4_ref,
          out_ref, tile_ref):
    base = pl.program_id(0) * _BB
    iota8 = lax.broadcasted_iota(jnp.int32, (8, _E), 0)

    for o in range(_BB // 8):
        acc = None
        for k in range(8):
            idx = idx_ref[base + o * 8 + k]
            cbase = pl.multiple_of((idx >> 3) << 3, 8)
            chunk = emb_ref[pl.ds(cbase, 8), :]
            shifted = pltpu.roll(chunk, k - (idx & 7), axis=0)
            acc = shifted if acc is None else jnp.where(iota8 == k, shifted, acc)
        tile_ref[o * 8:(o + 1) * 8, :] = acc

    v = lambda r: v64_ref[r:r + 1, :]
    if_b, ca_ib, ca_ob, fusion_b = v(0), v(1), v(2), v(3)
    msa_g, msa_be = v(4), v(5)
    fi1_g, fi1_be, fi1_b2 = v(6), v(7), v(8)
    fi2_g, fi2_be, fi2_b2 = v(9), v(10), v(11)
    on_g, on_be = v(12), v(13)
    s3, c3, w4 = v(14), v(15), v(16)
    fi1_b1, s1, c1 = v256_ref[0:1, :], v256_ref[1:2, :], v256_ref[2:3, :]
    fi2_b1, s2, c2 = v128_ref[0:1, :], v128_ref[1:2, :], v128_ref[2:3, :]

    emb = tile_ref[...]
    feat = feat_ref[...]
    iflin = jnp.dot(feat, ifw_ref[...], preferred_element_type=_F32) + if_b
    kv = (jnp.dot(emb.astype(_BF16), cae_ref[...], preferred_element_type=_F32)
          + _dot(iflin, caf_ref[...]) + ca_ib)
    ca = _dot(kv, cao_ref[...]) + ca_ob
    fused = jnp.broadcast_to(fusion_b, ca.shape)
    for i in range(3):
        sl = slice(i * _E, (i + 1) * _E)
        vi = _dot(ca, vw_ref[sl, :]) + v(17 + i)
        oi = _dot(vi, ow_ref[sl, :]) + v(20 + i)
        fused = fused + _dot(oi, fuw_ref[sl, :])
    x = _ln(fused + ca, msa_g, msa_be)
    h = _ln(x, fi1_g, fi1_be)
    h = jax.nn.relu(_dot(h, fi1w1_ref[...]) + fi1_b1)
    h = jax.nn.relu(_dot(h, fi1w2_ref[...]) + fi1_b2)
    x = x + h
    h = _ln(x, fi2_g, fi2_be)
    h = jax.nn.relu(_dot(h, fi2w1_ref[...]) + fi2_b1)
    h = jax.nn.relu(_dot(h, fi2w2_ref[...]) + fi2_b2)
    x = x + h
    x = _ln(x, on_g, on_be)
    y = jax.nn.relu(_dot(x, pw1_ref[...]) * s1 + c1)
    y = jax.nn.relu(_dot(y, pw2_ref[...]) * s2 + c2)
    y = jax.nn.relu(_dot(y, pw3_ref[...]) * s3 + c3)
    yb = y.astype(_BF16).astype(_F32)
    w4b = w4.astype(_BF16).astype(_F32)
    out_ref[...] = jnp.sum(yb * w4b, axis=-1, keepdims=True) + c4_ref[0, 0]


@jax.jit
def kernel(user_idx, user_features, user_color_idx, user_size_idx,
           item_idx, item_features, params):
    del user_idx, user_features, user_color_idx, user_size_idx
    p = params

    def bnf(bg, bb, bm, bv):
        s = bg * lax.rsqrt(bv + _BN_EPS)
        return s, (0.0 - bm) * s + bb

    s1, c1 = bnf(p['bn1_g'], p['bn1_b'], p['bn1_m'], p['bn1_v'])
    s2, c2 = bnf(p['bn2_g'], p['bn2_b'], p['bn2_m'], p['bn2_v'])
    s3, c3 = bnf(p['bn3_g'], p['bn3_b'], p['bn3_m'], p['bn3_v'])
    c1 = p['p_b1'] * s1 + c1
    c2 = p['p_b2'] * s2 + c2
    c3 = p['p_b3'] * s3 + c3

    z64 = jnp.zeros((_E,), _F32)
    v64 = jnp.stack([
        p['if_b'], p['ca_ib'], p['ca_ob'], p['fusion_b'],
        p['msa_g'], p['msa_be'],
        p['fi1_g'], p['fi1_be'], p['fi1_b2'],
        p['fi2_g'], p['fi2_be'], p['fi2_b2'],
        p['on_g'], p['on_be'],
        s3, c3, p['p_W4'][0],
        p['msa_inb'][0][2 * _E:], p['msa_inb'][1][2 * _E:], p['msa_inb'][2][2 * _E:],
        p['msa_ob'][0], p['msa_ob'][1], p['msa_ob'][2], z64,
    ], axis=0)
    v256 = jnp.stack([p['fi1_b1'], s1, c1], axis=0)
    v128 = jnp.stack([p['fi2_b1'], s2, c2], axis=0)
    c4 = p['p_b4'].reshape(1, 1)

    bf = lambda a: a.astype(_BF16)
    vw = jnp.concatenate([p['msa_inW'][i][2 * _E:, :].T for i in range(3)], axis=0)
    ow = jnp.concatenate([p['msa_oW'][i].T for i in range(3)], axis=0)
    fuw = jnp.concatenate([p['fusion_W'][:, i * _E:(i + 1) * _E].T for i in range(3)], axis=0)

    idx = item_idx.astype(jnp.int32)
    emb = p['item_emb']
    feat = bf(item_features)

    nb = _B // _BB
    cm = lambda i, s: (0, 0)
    wspec = lambda: pl.BlockSpec(None, cm)
    grid_spec = pltpu.PrefetchScalarGridSpec(
        num_scalar_prefetch=1,
        grid=(nb,),
        in_specs=[
            pl.BlockSpec((emb.shape[0], _E), cm),
            pl.BlockSpec((_BB, 128), lambda i, s: (i, 0)),
            wspec(), wspec(), wspec(), wspec(),
            wspec(), wspec(), wspec(),
            wspec(), wspec(), wspec(), wspec(),
            wspec(), wspec(), wspec(),
            wspec(), wspec(), wspec(), wspec(),
        ],
        out_specs=pl.BlockSpec((_BB, 1), lambda i, s: (i, 0)),
        scratch_shapes=[pltpu.VMEM((_BB, _E), _F32)],
    )
    return pl.pallas_call(
        _body,
        grid_spec=grid_spec,
        out_shape=jax.ShapeDtypeStruct((_B, 1), _F32),
        compiler_params=pltpu.CompilerParams(
            dimension_semantics=("parallel",),
        ),
    )(idx, emb, feat,
      bf(p['if_W'].T), bf(p['ca_iW'][:, :_E].T), bf(p['ca_iW'][:, _E:].T), bf(p['ca_oW'].T),
      bf(vw), bf(ow), bf(fuw),
      bf(p['fi1_W1'].T), bf(p['fi1_W2'].T), bf(p['fi2_W1'].T), bf(p['fi2_W2'].T),
      bf(p['p_W1'].T), bf(p['p_W2'].T), bf(p['p_W3'].T),
      v64, v256, v128, c4)


# speedup vs baseline: 3.6436x; 1.0743x over previous
import jax
import jax.numpy as jnp
from jax import lax
from jax.experimental import pallas as pl
from jax.experimental.pallas import tpu as pltpu

_B = 32768
_E = 64
_BB = 512
_LN_EPS = 1e-5
_BN_EPS = 1e-5

_F32 = jnp.float32
_BF16 = jnp.bfloat16

_DN = (((1,), (1,)), ((), ()))


def _ln(x, g, b):
    m = x.mean(-1, keepdims=True)
    v = jnp.var(x, axis=-1, keepdims=True)
    return (x - m) * lax.rsqrt(v + _LN_EPS) * g + b


def _dgt(x, w):
    return lax.dot_general(x.astype(_BF16), w, _DN,
                           preferred_element_type=_F32)


def _body(idx_ref, emb_ref, feat_ref,
          ifw_ref, caiw_ref, caow_ref,
          inw0_ref, inw1_ref, inw2_ref,
          ow0_ref, ow1_ref, ow2_ref, fusw_ref,
          fi1w1_ref, fi1w2_ref, fi2w1_ref, fi2w2_ref,
          pw1_ref, pw2_ref, pw3_ref, pw4_ref,
          if_b_ref, ca_ib_ref, ca_ob_ref,
          inb0_ref, inb1_ref, inb2_ref,
          ob0_ref, ob1_ref, ob2_ref, fus_b_ref,
          msa_g_ref, msa_be_ref,
          fi1_b1_ref, fi1_b2_ref, fi1_g_ref, fi1_be_ref,
          fi2_b1_ref, fi2_b2_ref, fi2_g_ref, fi2_be_ref,
          on_g_ref, on_be_ref,
          pb1_ref, bn1g_ref, bn1b_ref, bn1m_ref, bn1v_ref,
          pb2_ref, bn2g_ref, bn2b_ref, bn2m_ref, bn2v_ref,
          pb3_ref, bn3g_ref, bn3b_ref, bn3m_ref, bn3v_ref,
          pb4_ref,
          out_ref, tile_ref):
    base = pl.program_id(0) * _BB
    iota8 = lax.broadcasted_iota(jnp.int32, (8, _E), 0)

    for o in range(_BB // 8):
        acc = None
        for k in range(8):
            idx = idx_ref[base + o * 8 + k]
            cbase = pl.multiple_of((idx >> 3) << 3, 8)
            chunk = emb_ref[pl.ds(cbase, 8), :]
            shifted = pltpu.roll(chunk, k - (idx & 7), axis=0)
            acc = shifted if acc is None else jnp.where(iota8 == k, shifted, acc)
        tile_ref[o * 8:(o + 1) * 8, :] = acc

    bf = lambda r: r[...].astype(_BF16)
    emb = tile_ref[...]
    feat = feat_ref[...]
    iflin = _dgt(feat, bf(ifw_ref)) + if_b_ref[...]
    caiw = bf(caiw_ref)
    kv = (_dgt(emb, caiw[:, :_E]) + _dgt(iflin, caiw[:, _E:]) + ca_ib_ref[...])
    ca = _dgt(kv, bf(caow_ref)) + ca_ob_ref[...]
    fused = jnp.broadcast_to(fus_b_ref[...], ca.shape)
    fusw = bf(fusw_ref)
    for i, (inw_ref, inb_ref, ow_ref, ob_ref) in enumerate((
            (inw0_ref, inb0_ref, ow0_ref, ob0_ref),
            (inw1_ref, inb1_ref, ow1_ref, ob1_ref),
            (inw2_ref, inb2_ref, ow2_ref, ob2_ref))):
        vi = _dgt(ca, bf(inw_ref)[2 * _E:, :]) + inb_ref[:, 2 * _E:]
        oi = _dgt(vi, bf(ow_ref)) + ob_ref[...]
        fused = fused + _dgt(oi, fusw[:, i * _E:(i + 1) * _E])
    x = _ln(fused + ca, msa_g_ref[...], msa_be_ref[...])
    h = _ln(x, fi1_g_ref[...], fi1_be_ref[...])
    h = jax.nn.relu(_dgt(h, bf(fi1w1_ref)) + fi1_b1_ref[...])
    h = jax.nn.relu(_dgt(h, bf(fi1w2_ref)) + fi1_b2_ref[...])
    x = x + h
    h = _ln(x, fi2_g_ref[...], fi2_be_ref[...])
    h = jax.nn.relu(_dgt(h, bf(fi2w1_ref)) + fi2_b1_ref[...])
    h = jax.nn.relu(_dgt(h, bf(fi2w2_ref)) + fi2_b2_ref[...])
    x = x + h
    x = _ln(x, on_g_ref[...], on_be_ref[...])
    def bn(y_lin, pb, g, b, m, v):
        s = g[...] * lax.rsqrt(v[...] + _BN_EPS)
        return y_lin * s + ((pb[...] - m[...]) * s + b[...])
    y = jax.nn.relu(bn(_dgt(x, bf(pw1_ref)), pb1_ref, bn1g_ref, bn1b_ref, bn1m_ref, bn1v_ref))
    y = jax.nn.relu(bn(_dgt(y, bf(pw2_ref)), pb2_ref, bn2g_ref, bn2b_ref, bn2m_ref, bn2v_ref))
    y = jax.nn.relu(bn(_dgt(y, bf(pw3_ref)), pb3_ref, bn3g_ref, bn3b_ref, bn3m_ref, bn3v_ref))
    yb = y.astype(_BF16).astype(_F32)
    w4b = pw4_ref[...].astype(_BF16).astype(_F32)
    out_ref[...] = jnp.sum(yb * w4b, axis=-1, keepdims=True) + pb4_ref[0, 0]


@jax.jit
def kernel(user_idx, user_features, user_color_idx, user_size_idx,
           item_idx, item_features, params):
    del user_idx, user_features, user_color_idx, user_size_idx
    p = params
    r = lambda a: a.reshape(1, -1)

    nb = _B // _BB
    vmem = lambda: pl.BlockSpec(memory_space=pltpu.MemorySpace.VMEM)
    n_resident = 1 + 18 + 38
    grid_spec = pltpu.PrefetchScalarGridSpec(
        num_scalar_prefetch=1,
        grid=(nb,),
        in_specs=[
            vmem(),
            pl.BlockSpec((_BB, 128), lambda i, s: (i, 0)),
        ] + [vmem() for _ in range(n_resident - 1)],
        out_specs=pl.BlockSpec((_BB, 1), lambda i, s: (i, 0)),
        scratch_shapes=[pltpu.VMEM((_BB, _E), _F32)],
    )
    return pl.pallas_call(
        _body,
        grid_spec=grid_spec,
        out_shape=jax.ShapeDtypeStruct((_B, 1), _F32),
        compiler_params=pltpu.CompilerParams(
            dimension_semantics=("parallel",),
        ),
    )(item_idx.astype(jnp.int32), p['item_emb'], item_features,
      p['if_W'], p['ca_iW'], p['ca_oW'],
      p['msa_inW'][0], p['msa_inW'][1], p['msa_inW'][2],
      p['msa_oW'][0], p['msa_oW'][1], p['msa_oW'][2], p['fusion_W'],
      p['fi1_W1'], p['fi1_W2'], p['fi2_W1'], p['fi2_W2'],
      p['p_W1'], p['p_W2'], p['p_W3'], p['p_W4'],
      r(p['if_b']), r(p['ca_ib']), r(p['ca_ob']),
      r(p['msa_inb'][0]), r(p['msa_inb'][1]), r(p['msa_inb'][2]),
      r(p['msa_ob'][0]), r(p['msa_ob'][1]), r(p['msa_ob'][2]), r(p['fusion_b']),
      r(p['msa_g']), r(p['msa_be']),
      r(p['fi1_b1']), r(p['fi1_b2']), r(p['fi1_g']), r(p['fi1_be']),
      r(p['fi2_b1']), r(p['fi2_b2']), r(p['fi2_g']), r(p['fi2_be']),
      r(p['on_g']), r(p['on_be']),
      r(p['p_b1']), r(p['bn1_g']), r(p['bn1_b']), r(p['bn1_m']), r(p['bn1_v']),
      r(p['p_b2']), r(p['bn2_g']), r(p['bn2_b']), r(p['bn2_m']), r(p['bn2_v']),
      r(p['p_b3']), r(p['bn3_g']), r(p['bn3_b']), r(p['bn3_m']), r(p['bn3_v']),
      r(p['p_b4']))
